# Initial kernel scaffold; baseline (speedup 1.0000x reference)
import jax
import jax.numpy as jnp
from jax.experimental import pallas as pl


def kernel(x, weight_fp):
    raise NotImplementedError("write your pallas kernel here")


# speedup vs baseline: 1.0265x; 1.0265x over previous
import functools

import jax
import jax.numpy as jnp
from jax import lax
from jax.experimental import pallas as pl
from jax.experimental.pallas import tpu as pltpu
from jax.experimental.pallas import tpu_sc as plsc

NUM_EMB = 1000000
DIM = 32
ROWS128 = NUM_EMB * DIM // 128
BLK = 2000
N_BLOCKS = ROWS128 // BLK

B = 16384 * 50
NW = 32
B_PER_W = B // NW
CH = 1024
K = CH // 128
N_CHUNKS = B_PER_W // CH


def _maxabs_body(x_ref, o_ref):
    i = pl.program_id(0)

    @pl.when(i == 0)
    def _init():
        o_ref[...] = jnp.zeros((1, 1), jnp.float32)

    o_ref[...] = jnp.maximum(o_ref[...], jnp.max(jnp.abs(x_ref[...])))


def _quant_body(s_ref, x_ref, o_ref):
    scale = jnp.maximum(s_ref[...] / 7.0, 1e-08)
    o_ref[...] = jnp.clip(jnp.round(x_ref[...] / scale), -8.0, 7.0) * scale


@functools.cache
def _make_gather():
    mesh = plsc.VectorSubcoreMesh(core_axis_name="c", subcore_axis_name="s")

    @functools.partial(
        pl.kernel,
        mesh=mesh,
        compiler_params=pltpu.CompilerParams(use_tc_tiling_on_sc=False),
        out_type=jax.ShapeDtypeStruct((B, DIM), jnp.float32),
        scratch_types=[
            pltpu.VMEM((K, 128), jnp.int32),
            pltpu.VMEM((CH, DIM), jnp.float32),
            pltpu.SemaphoreType.DMA,
        ],
    )
    def gather_k(table_hbm, idx_hbm, out_hbm, idx_v, rows_v, sem):
        wid = lax.axis_index("s") * 2 + lax.axis_index("c")
        row_base = wid * (B_PER_W // 128)

        def chunk(g, _):
            idx_row = row_base + g * K
            pltpu.sync_copy(idx_hbm.at[pl.ds(idx_row, K)], idx_v)
            for j in range(K):
                pltpu.async_copy(
                    table_hbm.at[idx_v.at[j]],
                    rows_v.at[pl.ds(j * 128, 128)],
                    sem,
                )
            for j in range(K):
                pltpu.make_async_copy(
                    table_hbm.at[idx_v.at[j]],
                    rows_v.at[pl.ds(j * 128, 128)],
                    sem,
                ).wait()
            pltpu.sync_copy(
                rows_v, out_hbm.at[pl.ds(wid * B_PER_W + g * CH, CH)]
            )
            return 0

        lax.fori_loop(0, N_CHUNKS, chunk, 0)

    return gather_k


def kernel(x, weight_fp):
    w128 = weight_fp.reshape(ROWS128, 128)

    maxabs = pl.pallas_call(
        _maxabs_body,
        grid=(N_BLOCKS,),
        in_specs=[pl.BlockSpec((BLK, 128), lambda i: (i, 0))],
        out_specs=pl.BlockSpec((1, 1), lambda i: (0, 0)),
        out_shape=jax.ShapeDtypeStruct((1, 1), jnp.float32),
    )(w128)

    w_q = pl.pallas_call(
        _quant_body,
        grid=(N_BLOCKS,),
        in_specs=[
            pl.BlockSpec((1, 1), lambda i: (0, 0)),
            pl.BlockSpec((BLK, 128), lambda i: (i, 0)),
        ],
        out_specs=pl.BlockSpec((BLK, 128), lambda i: (i, 0)),
        out_shape=jax.ShapeDtypeStruct((ROWS128, 128), jnp.float32),
    )(maxabs, w128)

    w_q = w_q.reshape(NUM_EMB, DIM)
    idx = x.reshape(B // 128, 128).astype(jnp.int32)
    out = _make_gather()(w_q, idx)
    return out.reshape(x.shape[0], x.shape[1], DIM)


# speedup vs baseline: 1.5254x; 1.4861x over previous
import functools

import jax
import jax.numpy as jnp
from jax import lax
from jax.experimental import pallas as pl
from jax.experimental.pallas import tpu as pltpu
from jax.experimental.pallas import tpu_sc as plsc

NUM_EMB = 1000000
DIM = 32
ROWS128 = NUM_EMB * DIM // 128
BLK = 2000
N_BLOCKS = ROWS128 // BLK

BATCH = 16384
HIST = 50
NW = 32
B_PER_W = BATCH // NW
NB = 16
N_CHUNKS = B_PER_W // NB


def _maxabs_body(x_ref, o_ref):
    i = pl.program_id(0)

    @pl.when(i == 0)
    def _init():
        o_ref[...] = jnp.zeros((1, 1), jnp.float32)

    o_ref[...] = jnp.maximum(o_ref[...], jnp.max(jnp.abs(x_ref[...])))


def _quant_body(s_ref, x_ref, o_ref):
    scale = jnp.maximum(s_ref[...] / 7.0, 1e-08)
    o_ref[...] = jnp.clip(jnp.round(x_ref[...] / scale), -8.0, 7.0) * scale


@functools.cache
def _make_gather():
    mesh = plsc.VectorSubcoreMesh(core_axis_name="c", subcore_axis_name="s")

    @functools.partial(
        pl.kernel,
        mesh=mesh,
        compiler_params=pltpu.CompilerParams(use_tc_tiling_on_sc=False),
        out_type=jax.ShapeDtypeStruct((BATCH, HIST, DIM), jnp.float32),
        scratch_types=[
            pltpu.VMEM((NB, HIST), jnp.int32),
            pltpu.VMEM((NB, HIST, DIM), jnp.float32),
            pltpu.SemaphoreType.DMA,
        ],
    )
    def gather_k(table_hbm, idx_hbm, out_hbm, idx_v, rows_v, sem):
        wid = lax.axis_index("s") * 2 + lax.axis_index("c")
        base = wid * B_PER_W

        def chunk(g, _):
            i0 = base + g * NB
            pltpu.sync_copy(idx_hbm.at[pl.ds(i0, NB)], idx_v)
            for b in range(NB):
                pltpu.async_copy(
                    table_hbm.at[idx_v.at[b]], rows_v.at[b], sem
                )
            for b in range(NB):
                pltpu.make_async_copy(
                    table_hbm.at[idx_v.at[b]], rows_v.at[b], sem
                ).wait()
            pltpu.sync_copy(rows_v, out_hbm.at[pl.ds(i0, NB)])
            return 0

        lax.fori_loop(0, N_CHUNKS, chunk, 0)

    return gather_k


def kernel(x, weight_fp):
    w128 = weight_fp.reshape(ROWS128, 128)

    maxabs = pl.pallas_call(
        _maxabs_body,
        grid=(N_BLOCKS,),
        in_specs=[pl.BlockSpec((BLK, 128), lambda i: (i, 0))],
        out_specs=pl.BlockSpec((1, 1), lambda i: (0, 0)),
        out_shape=jax.ShapeDtypeStruct((1, 1), jnp.float32),
    )(w128)

    w_q = pl.pallas_call(
        _quant_body,
        grid=(N_BLOCKS,),
        in_specs=[
            pl.BlockSpec((1, 1), lambda i: (0, 0)),
            pl.BlockSpec((BLK, 128), lambda i: (i, 0)),
        ],
        out_specs=pl.BlockSpec((BLK, 128), lambda i: (i, 0)),
        out_shape=jax.ShapeDtypeStruct((ROWS128, 128), jnp.float32),
    )(maxabs, w128)

    w_q = w_q.reshape(NUM_EMB, DIM)
    return _make_gather()(w_q, x.astype(jnp.int32))
